# baseline (speedup 1.0000x reference)
import functools

import jax
import jax.numpy as jnp
from jax import lax
from jax.experimental import pallas as pl
from jax.experimental.pallas import tpu as pltpu
from jax.experimental.pallas import tpu_sc as plsc

EPS = 1e-06

N_ROWS = 16384
N_COLS = 512
N_BOUND = 128
NW = 32
ROWS_PER_WORKER = N_ROWS // NW
CHUNK_ROWS = 64
N_CHUNKS = ROWS_PER_WORKER // CHUNK_ROWS
NBUF = 3
JVREGS = N_BOUND // 16


def _sc_body(x_hbm, idx_hbm, ub_hbm, out_hbm, idx_v, ub_v, bufs, sin, sout):
    wid = lax.axis_index("c") * 16 + lax.axis_index("s")
    base = wid * ROWS_PER_WORKER

    pltpu.sync_copy(idx_hbm, idx_v)
    pltpu.sync_copy(ub_hbm, ub_v)
    colv = [idx_v[pl.ds(16 * j, 16)] for j in range(JVREGS)]
    ub = ub_v[...]
    zero16 = jnp.zeros((16,), jnp.int32)

    def copy_in(g, b):
        return pltpu.make_async_copy(
            x_hbm.at[pl.ds(base + g * CHUNK_ROWS, CHUNK_ROWS), :],
            bufs[b], sin[b])

    def copy_out(g, b):
        return pltpu.make_async_copy(
            bufs[b],
            out_hbm.at[pl.ds(base + g * CHUNK_ROWS, CHUNK_ROWS), :], sout[b])

    def compute(b):
        buf = bufs[b]

        @plsc.parallel_loop(0, CHUNK_ROWS, unroll=1)
        def row(r):
            rv = zero16 + r
            xs = [plsc.load_gather(buf, [rv, colv[j]]) for j in range(JVREGS)]
            ys = []
            for x in xs:
                e = jnp.exp(-jnp.abs(x))
                z = e / (2.0 + e)
                z2 = z * z
                lg = z * (2.0 + z2 * (2.0 / 3.0 + z2 * 0.4))
                ys.append(ub + (jnp.minimum(x, 0.0) - lg))
            for j, y in enumerate(ys):
                plsc.store_scatter(buf, [rv, colv[j]], y)

    for b in range(NBUF - 1):
        copy_in(b, b).start()
    for g in range(N_CHUNKS):
        b = g % NBUF
        copy_in(g, b).wait()
        compute(b)
        copy_out(g, b).start()
        ng = g + NBUF - 1
        if ng < N_CHUNKS:
            nb = ng % NBUF
            if ng >= NBUF:
                copy_out(ng - NBUF, nb).wait()
            copy_in(ng, nb).start()
    for g in range(N_CHUNKS - NBUF, N_CHUNKS):
        copy_out(g, g % NBUF).wait()


def kernel(x, bounded_col_idx, upper_bounds):
    n_rows, n_cols = x.shape
    idx32 = bounded_col_idx.astype(jnp.int32)
    ub_vec = jnp.full((16,), jnp.asarray(upper_bounds, jnp.float32) - EPS,
                      jnp.float32)
    mesh = plsc.VectorSubcoreMesh(core_axis_name="c", subcore_axis_name="s")
    f = functools.partial(
        pl.kernel,
        out_type=jax.ShapeDtypeStruct((n_rows, n_cols), jnp.float32),
        mesh=mesh,
        compiler_params=pltpu.CompilerParams(needs_layout_passes=False),
        scratch_types=(
            [pltpu.VMEM((N_BOUND,), jnp.int32),
             pltpu.VMEM((16,), jnp.float32),
             [pltpu.VMEM((CHUNK_ROWS, N_COLS), jnp.float32)
              for _ in range(NBUF)],
             [pltpu.SemaphoreType.DMA for _ in range(NBUF)],
             [pltpu.SemaphoreType.DMA for _ in range(NBUF)]]
        ),
    )(_sc_body)
    return f(x, idx32, ub_vec)


# speedup vs baseline: 1.7499x; 1.7499x over previous
import jax
import jax.numpy as jnp
from jax.experimental import pallas as pl

EPS = 1e-06
ROWGROUPS_PER_BLOCK = 512
CHUNK = 64

_LOG2E = 1.4426950408889634
_LN2 = 0.6931471805599453


def _clamp_kernel(mask_ref, ub_ref, x_ref, o_ref):
    m = mask_ref[...] > 0.5
    ub = ub_ref[0, 0]

    def body(k, _):
        x = x_ref[pl.ds(k * CHUNK, CHUNK)]
        a = jnp.abs(x)
        e = jnp.exp2(a * (-_LOG2E))
        ls = jnp.minimum(x, 0.0) - _LN2 * jnp.log2(1.0 + e)
        o_ref[pl.ds(k * CHUNK, CHUNK)] = jnp.where(m, ub + ls, x)
        return 0

    jax.lax.fori_loop(0, ROWGROUPS_PER_BLOCK // CHUNK, body, 0, unroll=False)


def kernel(x, bounded_col_idx, upper_bounds):
    n_rows, n_cols = x.shape
    x3 = x.reshape(n_rows // 8, 8, n_cols)
    grid = (x3.shape[0] // ROWGROUPS_PER_BLOCK,)
    mask = jnp.zeros((n_cols,), jnp.float32).at[bounded_col_idx].set(1.0)
    mask3 = jnp.broadcast_to(mask, (1, 8, n_cols))
    ub2d = (jnp.asarray(upper_bounds, jnp.float32) - EPS).reshape(1, 1)
    out = pl.pallas_call(
        _clamp_kernel,
        grid=grid,
        in_specs=[
            pl.BlockSpec((1, 8, n_cols), lambda i: (0, 0, 0)),
            pl.BlockSpec((1, 1), lambda i: (0, 0)),
            pl.BlockSpec((ROWGROUPS_PER_BLOCK, 8, n_cols), lambda i: (i, 0, 0)),
        ],
        out_specs=pl.BlockSpec((ROWGROUPS_PER_BLOCK, 8, n_cols), lambda i: (i, 0, 0)),
        out_shape=jax.ShapeDtypeStruct(x3.shape, x.dtype),
    )(mask3, ub2d, x3)
    return out.reshape(n_rows, n_cols)
